# baseline (speedup 1.0000x reference)
import functools
import math

import jax
import jax.numpy as jnp
import numpy as np
from jax import lax
from jax.experimental import pallas as pl
from jax.experimental.pallas import tpu as pltpu
from jax.experimental.pallas import tpu_sc as plsc

_D_MODEL = 768
_N_EXPERTS = 64
_TOP_K = 2
_CAPACITY_FACTOR = 1.2
_N_TOKENS = 32768
_NK = _N_TOKENS * _TOP_K
_CAP = int(math.ceil(_CAPACITY_FACTOR * _N_TOKENS / _N_EXPERTS))

_TOK_BLK = 1024
_CS = 4096
_NCH = _N_TOKENS // _CS
_V = 8
_BIG = np.int32(1 << 30)
_HI0 = 0x3F800001


def _router_body(x_ref, w_ref, b_ref, g1_ref, g2_ref, i1_ref, i2_ref, imp_ref):
    logits = jax.lax.dot_general(
        w_ref[...], x_ref[...], (((1,), (1,)), ((), ())),
        preferred_element_type=jnp.float32)
    logits = logits + b_ref[...]
    m = jnp.max(logits, axis=0, keepdims=True)
    e = jnp.exp(logits - m)
    p = e / jnp.sum(e, axis=0, keepdims=True)

    eidx = jax.lax.broadcasted_iota(jnp.int32, p.shape, 0)
    m1 = jnp.max(p, axis=0, keepdims=True)
    i1 = jnp.min(jnp.where(p == m1, eidx, _N_EXPERTS), axis=0, keepdims=True)
    p2 = jnp.where(eidx == i1, -1.0, p)
    m2 = jnp.max(p2, axis=0, keepdims=True)
    i2 = jnp.min(jnp.where(p2 == m2, eidx, _N_EXPERTS), axis=0, keepdims=True)

    g1_ref[...] = m1
    g2_ref[...] = m2
    i1_ref[...] = i1
    i2_ref[...] = i2

    @pl.when(pl.program_id(0) == 0)
    def _():
        imp_ref[...] = jnp.zeros_like(imp_ref)

    imp_ref[...] += jnp.sum(p, axis=1, keepdims=True)


def _sc_capacity_body(g1_hbm, g2_hbm, i1_hbm, i2_hbm, imp_hbm,
                      k1_hbm, k2_hbm, aux_hbm,
                      gst, ist, gst2, ist2, buf, tbl_v, row_v, imp_v,
                      kst, aux_v, tbl_sh,
                      sem_g0, sem_g1, sem_i0, sem_i1):
    cid = lax.axis_index("c")
    sid = lax.axis_index("s")
    wid = sid * 2 + cid
    cap = jnp.int32(_CAP)
    lane16 = lax.iota(jnp.int32, 16)
    gsems = (sem_g0, sem_g1)
    isems = (sem_i0, sem_i1)

    def pcount(mask):
        return plsc.all_reduce_population_count(mask)[0]

    seq = [(g1_hbm, i1_hbm, c) for c in range(_NCH)] + \
          [(g2_hbm, i2_hbm, c) for c in range(_NCH)]

    def _start(entry, b):
        gh, ih, c = entry
        pltpu.make_async_copy(gh.at[pl.ds(c * _CS, _CS)],
                              gst.at[pl.ds(b * _CS, _CS)], gsems[b]).start()
        pltpu.make_async_copy(ih.at[pl.ds(c * _CS, _CS)],
                              ist.at[pl.ds(b * _CS, _CS)], isems[b]).start()

    def _wait(entry, b):
        gh, ih, c = entry
        pltpu.make_async_copy(gh.at[pl.ds(c * _CS, _CS)],
                              gst.at[pl.ds(b * _CS, _CS)], gsems[b]).wait()
        pltpu.make_async_copy(ih.at[pl.ds(c * _CS, _CS)],
                              ist.at[pl.ds(b * _CS, _CS)], isems[b]).wait()

    def count_pass(es):
        iseq = [(i1_hbm, c) for c in range(_NCH)] + \
               [(i2_hbm, c) for c in range(_NCH)]

        def _istart(entry, b):
            ih, c = entry
            pltpu.make_async_copy(ih.at[pl.ds(c * _CS, _CS)],
                                  ist.at[pl.ds(b * _CS, _CS)], isems[b]).start()

        def _iwait(entry, b):
            ih, c = entry
            pltpu.make_async_copy(ih.at[pl.ds(c * _CS, _CS)],
                                  ist.at[pl.ds(b * _CS, _CS)], isems[b]).wait()

        accs = tuple(jnp.zeros((16,), jnp.int32) for _ in es)
        for idx, entry in enumerate(iseq):
            b = idx & 1
            if idx == 0:
                _istart(entry, 0)
            if idx + 1 < len(iseq):
                _istart(iseq[idx + 1], 1 - b)
            _iwait(entry, b)
            ibase = b * _CS

            def cgrp(i, a4):
                iv = ist[pl.ds(ibase + i * 16, 16)]
                return tuple(a + (iv == e).astype(jnp.int32)
                             for a, e in zip(a4, es))

            accs = lax.fori_loop(0, _CS // 16, cgrp, accs)
        return [jnp.sum(a) for a in accs]

    def compact_pass(es, bases):
        offs = tuple(bases)
        for idx, entry in enumerate(seq):
            b = idx & 1
            if idx == 0:
                _start(entry, 0)
            if idx + 1 < len(seq):
                _start(seq[idx + 1], 1 - b)
            _wait(entry, b)
            gbase = b * _CS

            def grp(gi, offs4):
                gs, ms, cs = [], [], []
                for v in range(_V):
                    sl = pl.ds(gbase + gi * 16 * _V + v * 16, 16)
                    gv = gst[sl]
                    iv = ist[sl]
                    gs.append(gv)
                    mv = [iv == e for e in es]
                    ms.append(mv)
                    cs.append([pcount(m) for m in mv])
                offs4 = list(offs4)
                for v in range(_V):
                    for k in range(len(es)):
                        plsc.store_compressed(
                            buf.at[pl.ds(offs4[k], 16)], gs[v], mask=ms[v][k])
                        offs4[k] = offs4[k] + cs[v][k]
                return tuple(offs4)

            offs = lax.fori_loop(0, _CS // (16 * _V), grp, offs)

    def count_pred(base, n, pred):
        nv = lax.shift_right_arithmetic(n + 15, 4)

        def bd(i, acc):
            v = buf[pl.ds(base + i * 16, 16)]
            valid = (lane16 + i * 16) < n
            return acc + jnp.where(valid & pred(v), 1, 0).astype(jnp.int32)

        acc = lax.fori_loop(0, nv, bd, jnp.zeros((16,), jnp.int32))
        return jnp.sum(acc)

    def select(base, n):
        over = n > cap

        def it(_, lh):
            lo, hi = lh
            mid = lax.shift_right_arithmetic(lo + hi + 1, 1)
            ok = count_pred(base, n, lambda v: v >= mid) >= cap
            return jnp.where(ok, mid, lo), jnp.where(ok, hi, mid - 1)

        lo, _ = lax.fori_loop(0, 31, it, (jnp.int32(0), jnp.int32(_HI0)))
        thr = jnp.where(over, lo, 0)
        ngt = count_pred(base, n, lambda v: v > thr)
        nge = count_pred(base, n, lambda v: v >= thr)
        m_rem = cap - ngt
        tie = over & ((nge - ngt) > m_rem)
        load = jnp.minimum(n, cap)
        return thr, m_rem, tie, load

    def tie_pass(es, ts, ms):
        ne = len(es)
        carry = tuple([_BIG * jnp.int32(1)] * ne + [jnp.int32(0)] * ne)
        for c in range(_NCH):
            pltpu.sync_copy(g1_hbm.at[pl.ds(c * _CS, _CS)], gst.at[pl.ds(0, _CS)])
            pltpu.sync_copy(i1_hbm.at[pl.ds(c * _CS, _CS)], ist.at[pl.ds(0, _CS)])
            pltpu.sync_copy(g2_hbm.at[pl.ds(c * _CS, _CS)], gst2.at[pl.ds(0, _CS)])
            pltpu.sync_copy(i2_hbm.at[pl.ds(c * _CS, _CS)], ist2.at[pl.ds(0, _CS)])

            def stp(i, cr):
                iacc = list(cr[:ne])
                cacc = list(cr[ne:])
                sl = pl.ds(i * 16, 16)
                gv0 = gst[sl]
                iv0 = ist[sl]
                gv1 = gst2[sl]
                iv1 = ist2[sl]
                j0 = 2 * (lane16 + (c * _CS + i * 16))
                j1 = j0 + 1

                def upd(e, t, m, acc_i, acc_c):
                    eq0 = (iv0 == e) & (gv0 == t)
                    eq1 = (iv1 == e) & (gv1 == t)
                    s0 = eq0.astype(jnp.int32)
                    s1 = eq1.astype(jnp.int32)
                    cs0 = plsc.cumsum(s0)
                    cs1 = plsc.cumsum(s1)
                    pre0 = cs0 - s0
                    pre1 = cs1 - s1
                    pos0 = pre0 + pre1 + 1
                    pos1 = pre0 + s0 + pre1 + 1
                    tgt = m - acc_c
                    sel0 = eq0 & (pos0 == tgt)
                    sel1 = eq1 & (pos1 == tgt)
                    js = jnp.minimum(jnp.min(jnp.where(sel0, j0, _BIG)),
                                     jnp.min(jnp.where(sel1, j1, _BIG)))
                    pc = pcount(eq0) + pcount(eq1)
                    cross = (acc_c < m) & ((acc_c + pc) >= m)
                    return jnp.where(cross, js, acc_i), acc_c + pc

                for k in range(ne):
                    iacc[k], cacc[k] = upd(es[k], ts[k], ms[k],
                                           iacc[k], cacc[k])
                return tuple(iacc + cacc)

            carry = lax.fori_loop(0, _CS // 16, stp, carry)
        return tuple(carry[:ne])

    e_base = sid * 4
    es = [e_base + k for k in range(4)]
    cnts = count_pass(es)
    bases = [jnp.int32(0)]
    for k in range(3):
        bases.append(bases[k] + cnts[k])
    compact_pass(es, bases)
    ts, mrems, ties, loads = [], [], [], []
    for k in range(4):
        t_k, m_k, tie_k, load_k = select(bases[k], cnts[k])
        ts.append(t_k)
        mrems.append(m_k)
        ties.append(tie_k)
        loads.append(load_k)
    ivals = lax.cond(
        ties[0] | ties[1] | ties[2] | ties[3],
        lambda: tie_pass(es, ts, mrems),
        lambda: tuple(_BIG * jnp.int32(1) for _ in range(4)))
    vals = [(ts[k], jnp.where(ties[k], ivals[k], _BIG), loads[k])
            for k in range(4)]

    row = jnp.zeros((16,), jnp.int32)
    for k in range(4):
        t_k, i_k, l_k = vals[k]
        row = jnp.where(lane16 == k, t_k, row)
        row = jnp.where(lane16 == k + 4, i_k, row)
        row = jnp.where(lane16 == k + 8, l_k, row)
    row_v[...] = row
    pltpu.sync_copy(row_v, tbl_sh.at[pl.ds(sid * 16, 16)])
    plsc.subcore_barrier()
    plsc.subcore_barrier()
    pltpu.sync_copy(tbl_sh, tbl_v)

    ntok = _N_TOKENS // 32
    tok0 = wid * ntok
    pltpu.sync_copy(g1_hbm.at[pl.ds(tok0, ntok)], gst.at[pl.ds(0, ntok)])
    pltpu.sync_copy(i1_hbm.at[pl.ds(tok0, ntok)], ist.at[pl.ds(0, ntok)])
    pltpu.sync_copy(g2_hbm.at[pl.ds(tok0, ntok)], gst2.at[pl.ds(0, ntok)])
    pltpu.sync_copy(i2_hbm.at[pl.ds(tok0, ntok)], ist2.at[pl.ds(0, ntok)])

    def cstep(i, _):
        sl = pl.ds(i * 16, 16)
        j0 = 2 * (lane16 + (tok0 + i * 16))

        def keep_of(gv, iv, jv):
            flat = (lax.shift_right_arithmetic(iv, 2) * 16
                    + jnp.bitwise_and(iv, 3))
            t = plsc.load_gather(tbl_v, [flat])
            ithr = plsc.load_gather(tbl_v, [flat + 4])
            return ((gv > t) | ((gv == t) & (jv <= ithr))).astype(jnp.int32)

        kst[sl] = keep_of(gst[sl], ist[sl], j0)
        kst[pl.ds(ntok + i * 16, 16)] = keep_of(gst2[sl], ist2[sl], j0 + 1)
        return 0

    lax.fori_loop(0, ntok // 16, cstep, 0)
    pltpu.sync_copy(kst.at[pl.ds(0, ntok)], k1_hbm.at[pl.ds(tok0, ntok)])
    pltpu.sync_copy(kst.at[pl.ds(ntok, ntok)], k2_hbm.at[pl.ds(tok0, ntok)])

    @pl.when((cid == 0) & (sid == 0))
    def _():
        pltpu.sync_copy(imp_hbm, imp_v)
        acc = jnp.zeros((16,), jnp.float32)
        for g4 in range(4):
            ev = lane16 + 16 * g4
            flat = (lax.shift_right_arithmetic(ev, 2) * 16
                    + jnp.bitwise_and(ev, 3) + 8)
            lv = plsc.load_gather(tbl_v, [flat])
            impv = imp_v[pl.ds(16 * g4, 16)]
            acc = acc + impv * lv.astype(jnp.float32)
        scale = _N_EXPERTS / (float(_N_TOKENS) * float(_N_TOKENS))
        aux_v[...] = acc * scale
        pltpu.sync_copy(aux_v, aux_hbm)


def _sc_capacity(g1b, g2b, i1v, i2v, imp):
    mesh = plsc.VectorSubcoreMesh(core_axis_name="c", subcore_axis_name="s")
    fn = pl.kernel(
        _sc_capacity_body,
        out_type=[
            jax.ShapeDtypeStruct((_N_TOKENS,), jnp.int32),
            jax.ShapeDtypeStruct((_N_TOKENS,), jnp.int32),
            jax.ShapeDtypeStruct((16,), jnp.float32),
        ],
        mesh=mesh,
        compiler_params=pltpu.CompilerParams(needs_layout_passes=False),
        scratch_types=[
            pltpu.VMEM((2 * _CS,), jnp.int32),
            pltpu.VMEM((2 * _CS,), jnp.int32),
            pltpu.VMEM((2 * _CS,), jnp.int32),
            pltpu.VMEM((2 * _CS,), jnp.int32),
            pltpu.VMEM((_NK + 64,), jnp.int32),
            pltpu.VMEM((256,), jnp.int32),
            pltpu.VMEM((16,), jnp.int32),
            pltpu.VMEM((_N_EXPERTS,), jnp.float32),
            pltpu.VMEM((2 * (_N_TOKENS // 32),), jnp.int32),
            pltpu.VMEM((16,), jnp.float32),
            pltpu.VMEM_SHARED((256,), jnp.int32),
            pltpu.SemaphoreType.DMA,
            pltpu.SemaphoreType.DMA,
            pltpu.SemaphoreType.DMA,
            pltpu.SemaphoreType.DMA,
        ],
    )
    return fn(g1b, g2b, i1v, i2v, imp)


@jax.jit
def kernel(x, W, b, training):
    del training
    n_blocks = _N_TOKENS // _TOK_BLK
    g1, g2, i1, i2, imp = pl.pallas_call(
        _router_body,
        grid=(n_blocks,),
        in_specs=[
            pl.BlockSpec((_TOK_BLK, _D_MODEL), lambda i: (i, 0)),
            pl.BlockSpec((_N_EXPERTS, _D_MODEL), lambda i: (0, 0)),
            pl.BlockSpec((_N_EXPERTS, 1), lambda i: (0, 0)),
        ],
        out_specs=[
            pl.BlockSpec((1, _TOK_BLK), lambda i: (0, i)),
            pl.BlockSpec((1, _TOK_BLK), lambda i: (0, i)),
            pl.BlockSpec((1, _TOK_BLK), lambda i: (0, i)),
            pl.BlockSpec((1, _TOK_BLK), lambda i: (0, i)),
            pl.BlockSpec((_N_EXPERTS, 1), lambda i: (0, 0)),
        ],
        out_shape=[
            jax.ShapeDtypeStruct((1, _N_TOKENS), jnp.float32),
            jax.ShapeDtypeStruct((1, _N_TOKENS), jnp.float32),
            jax.ShapeDtypeStruct((1, _N_TOKENS), jnp.int32),
            jax.ShapeDtypeStruct((1, _N_TOKENS), jnp.int32),
            jax.ShapeDtypeStruct((_N_EXPERTS, 1), jnp.float32),
        ],
        compiler_params=pltpu.CompilerParams(
            dimension_semantics=("arbitrary",)),
    )(x, W, b.reshape(_N_EXPERTS, 1))

    g1b = jax.lax.bitcast_convert_type(g1, jnp.int32).reshape(_N_TOKENS)
    g2b = jax.lax.bitcast_convert_type(g2, jnp.int32).reshape(_N_TOKENS)
    k1, k2, aux16 = _sc_capacity(
        g1b, g2b, i1.reshape(_N_TOKENS), i2.reshape(_N_TOKENS),
        imp.reshape(_N_EXPERTS))

    topk_ids = jnp.stack([i1[0], i2[0]], axis=1)
    topk_gates = jnp.stack([g1[0], g2[0]], axis=1)
    keep_mask = jnp.stack([k1, k2], axis=1) != 0
    return topk_ids, topk_gates, jnp.sum(aux16), keep_mask


# speedup vs baseline: 1.1111x; 1.1111x over previous
import functools
import math

import jax
import jax.numpy as jnp
import numpy as np
from jax import lax
from jax.experimental import pallas as pl
from jax.experimental.pallas import tpu as pltpu
from jax.experimental.pallas import tpu_sc as plsc

_D_MODEL = 768
_N_EXPERTS = 64
_TOP_K = 2
_CAPACITY_FACTOR = 1.2
_N_TOKENS = 32768
_NK = _N_TOKENS * _TOP_K
_CAP = int(math.ceil(_CAPACITY_FACTOR * _N_TOKENS / _N_EXPERTS))

_TOK_BLK = 1024
_CS = 4096
_NCH = _N_TOKENS // _CS
_V = 8
_BIG = np.int32(1 << 30)
_HI0 = 0x3F800001


def _router_body(x_ref, w_ref, b_ref, g1_ref, g2_ref, i1_ref, i2_ref, imp_ref):
    logits = jax.lax.dot_general(
        w_ref[...], x_ref[...], (((1,), (1,)), ((), ())),
        preferred_element_type=jnp.float32)
    logits = logits + b_ref[...]
    m = jnp.max(logits, axis=0, keepdims=True)
    e = jnp.exp(logits - m)
    p = e / jnp.sum(e, axis=0, keepdims=True)

    eidx = jax.lax.broadcasted_iota(jnp.int32, p.shape, 0)
    m1 = jnp.max(p, axis=0, keepdims=True)
    i1 = jnp.min(jnp.where(p == m1, eidx, _N_EXPERTS), axis=0, keepdims=True)
    p2 = jnp.where(eidx == i1, -1.0, p)
    m2 = jnp.max(p2, axis=0, keepdims=True)
    i2 = jnp.min(jnp.where(p2 == m2, eidx, _N_EXPERTS), axis=0, keepdims=True)

    g1_ref[...] = m1
    g2_ref[...] = m2
    i1_ref[...] = i1
    i2_ref[...] = i2

    @pl.when(pl.program_id(0) == 0)
    def _():
        imp_ref[...] = jnp.zeros_like(imp_ref)

    imp_ref[...] += jnp.sum(p, axis=1, keepdims=True)


def _sc_capacity_body(g1_hbm, g2_hbm, i1_hbm, i2_hbm, imp_hbm,
                      k1_hbm, k2_hbm, aux_hbm,
                      gst, ist, gst2, ist2, buf, tbl_v, row_v, imp_v,
                      kst, aux_v, tbl_sh,
                      sem_g0, sem_g1, sem_i0, sem_i1):
    cid = lax.axis_index("c")
    sid = lax.axis_index("s")
    wid = sid * 2 + cid
    cap = jnp.int32(_CAP)
    lane16 = lax.iota(jnp.int32, 16)
    gsems = (sem_g0, sem_g1)
    isems = (sem_i0, sem_i1)

    def pcount(mask):
        return plsc.all_reduce_population_count(mask)[0]

    seq = [(g1_hbm, i1_hbm, c) for c in range(_NCH)] + \
          [(g2_hbm, i2_hbm, c) for c in range(_NCH)]

    def _start(entry, b):
        gh, ih, c = entry
        pltpu.make_async_copy(gh.at[pl.ds(c * _CS, _CS)],
                              gst.at[pl.ds(b * _CS, _CS)], gsems[b]).start()
        pltpu.make_async_copy(ih.at[pl.ds(c * _CS, _CS)],
                              ist.at[pl.ds(b * _CS, _CS)], isems[b]).start()

    def _wait(entry, b):
        gh, ih, c = entry
        pltpu.make_async_copy(gh.at[pl.ds(c * _CS, _CS)],
                              gst.at[pl.ds(b * _CS, _CS)], gsems[b]).wait()
        pltpu.make_async_copy(ih.at[pl.ds(c * _CS, _CS)],
                              ist.at[pl.ds(b * _CS, _CS)], isems[b]).wait()

    def compact_pass(e_up, e_dn):
        off_up = jnp.int32(0)
        off_dn = jnp.int32(_NK + 48)
        for idx, entry in enumerate(seq):
            b = idx & 1
            if idx == 0:
                _start(entry, 0)
            if idx + 1 < len(seq):
                _start(seq[idx + 1], 1 - b)
            _wait(entry, b)
            gbase = b * _CS

            def grp(gi, carry):
                ou, od = carry
                gs, mus, mds, cus, cds = [], [], [], [], []
                for v in range(_V):
                    sl = pl.ds(gbase + gi * 16 * _V + v * 16, 16)
                    gv = gst[sl]
                    iv = ist[sl]
                    mu = iv == e_up
                    md = iv == e_dn
                    gs.append(gv)
                    mus.append(mu)
                    mds.append(md)
                    cus.append(pcount(mu))
                    cds.append(pcount(md))
                for v in range(_V):
                    plsc.store_compressed(
                        buf.at[pl.ds(ou, 16)], gs[v], mask=mus[v])
                    ou = ou + cus[v]
                for v in range(_V):
                    od = od - cds[v]
                    plsc.store_compressed(
                        buf.at[pl.ds(od, 16)], gs[v], mask=mds[v])
                return ou, od

            off_up, off_dn = lax.fori_loop(
                0, _CS // (16 * _V), grp, (off_up, off_dn))
        return off_up, off_dn

    def count_pred(base, n, pred):
        nv = lax.shift_right_arithmetic(n + 15, 4)

        def bd(i, acc):
            v = buf[pl.ds(base + i * 16, 16)]
            valid = (lane16 + i * 16) < n
            return acc + jnp.where(valid & pred(v), 1, 0).astype(jnp.int32)

        acc = lax.fori_loop(0, nv, bd, jnp.zeros((16,), jnp.int32))
        return jnp.sum(acc)

    def select(base, n):
        over = n > cap

        def it(_, lh):
            lo, hi = lh
            mid = lax.shift_right_arithmetic(lo + hi + 1, 1)
            ok = count_pred(base, n, lambda v: v >= mid) >= cap
            return jnp.where(ok, mid, lo), jnp.where(ok, hi, mid - 1)

        lo, _ = lax.fori_loop(0, 31, it, (jnp.int32(0), jnp.int32(_HI0)))
        thr = jnp.where(over, lo, 0)
        ngt = count_pred(base, n, lambda v: v > thr)
        nge = count_pred(base, n, lambda v: v >= thr)
        m_rem = cap - ngt
        tie = over & ((nge - ngt) > m_rem)
        load = jnp.minimum(n, cap)
        return thr, m_rem, tie, load

    def tie_pass(e0, t0, m0, e1, t1, m1):
        carry = (_BIG * jnp.int32(1), _BIG * jnp.int32(1),
                 jnp.int32(0), jnp.int32(0))
        for c in range(_NCH):
            pltpu.sync_copy(g1_hbm.at[pl.ds(c * _CS, _CS)], gst.at[pl.ds(0, _CS)])
            pltpu.sync_copy(i1_hbm.at[pl.ds(c * _CS, _CS)], ist.at[pl.ds(0, _CS)])
            pltpu.sync_copy(g2_hbm.at[pl.ds(c * _CS, _CS)], gst2.at[pl.ds(0, _CS)])
            pltpu.sync_copy(i2_hbm.at[pl.ds(c * _CS, _CS)], ist2.at[pl.ds(0, _CS)])

            def stp(i, cr):
                ia, ib, ca, cb = cr
                sl = pl.ds(i * 16, 16)
                gv0 = gst[sl]
                iv0 = ist[sl]
                gv1 = gst2[sl]
                iv1 = ist2[sl]
                j0 = 2 * (lane16 + (c * _CS + i * 16))
                j1 = j0 + 1

                def upd(e, t, m, acc_i, acc_c):
                    eq0 = (iv0 == e) & (gv0 == t)
                    eq1 = (iv1 == e) & (gv1 == t)
                    s0 = eq0.astype(jnp.int32)
                    s1 = eq1.astype(jnp.int32)
                    cs0 = plsc.cumsum(s0)
                    cs1 = plsc.cumsum(s1)
                    pre0 = cs0 - s0
                    pre1 = cs1 - s1
                    pos0 = pre0 + pre1 + 1
                    pos1 = pre0 + s0 + pre1 + 1
                    tgt = m - acc_c
                    sel0 = eq0 & (pos0 == tgt)
                    sel1 = eq1 & (pos1 == tgt)
                    js = jnp.minimum(jnp.min(jnp.where(sel0, j0, _BIG)),
                                     jnp.min(jnp.where(sel1, j1, _BIG)))
                    pc = pcount(eq0) + pcount(eq1)
                    cross = (acc_c < m) & ((acc_c + pc) >= m)
                    return jnp.where(cross, js, acc_i), acc_c + pc

                ia, ca = upd(e0, t0, m0, ia, ca)
                ib, cb = upd(e1, t1, m1, ib, cb)
                return ia, ib, ca, cb

            carry = lax.fori_loop(0, _CS // 16, stp, carry)
        return carry[0], carry[1]

    e_base = sid * 4
    vals = []
    for p in range(2):
        e_u = e_base + 2 * p
        e_d = e_u + 1
        n_up, off_dn = compact_pass(e_u, e_d)
        n_dn = jnp.int32(_NK + 48) - off_dn
        t0, m0, tie0, load0 = select(jnp.int32(0), n_up)
        t1, m1, tie1, load1 = select(off_dn, n_dn)
        i0, i1_ = lax.cond(
            tie0 | tie1,
            lambda: tie_pass(e_u, t0, m0, e_d, t1, m1),
            lambda: (_BIG * jnp.int32(1), _BIG * jnp.int32(1)))
        i0 = jnp.where(tie0, i0, _BIG)
        i1_ = jnp.where(tie1, i1_, _BIG)
        vals.append((t0, i0, load0))
        vals.append((t1, i1_, load1))

    row = jnp.zeros((16,), jnp.int32)
    for k in range(4):
        t_k, i_k, l_k = vals[k]
        row = jnp.where(lane16 == k, t_k, row)
        row = jnp.where(lane16 == k + 4, i_k, row)
        row = jnp.where(lane16 == k + 8, l_k, row)
    row_v[...] = row
    pltpu.sync_copy(row_v, tbl_sh.at[pl.ds(sid * 16, 16)])
    plsc.subcore_barrier()
    plsc.subcore_barrier()
    pltpu.sync_copy(tbl_sh, tbl_v)

    ntok = _N_TOKENS // 32
    tok0 = wid * ntok
    pltpu.sync_copy(g1_hbm.at[pl.ds(tok0, ntok)], gst.at[pl.ds(0, ntok)])
    pltpu.sync_copy(i1_hbm.at[pl.ds(tok0, ntok)], ist.at[pl.ds(0, ntok)])
    pltpu.sync_copy(g2_hbm.at[pl.ds(tok0, ntok)], gst2.at[pl.ds(0, ntok)])
    pltpu.sync_copy(i2_hbm.at[pl.ds(tok0, ntok)], ist2.at[pl.ds(0, ntok)])

    def cstep(i, _):
        sl = pl.ds(i * 16, 16)
        j0 = 2 * (lane16 + (tok0 + i * 16))

        def keep_of(gv, iv, jv):
            flat = (lax.shift_right_arithmetic(iv, 2) * 16
                    + jnp.bitwise_and(iv, 3))
            t = plsc.load_gather(tbl_v, [flat])
            ithr = plsc.load_gather(tbl_v, [flat + 4])
            return ((gv > t) | ((gv == t) & (jv <= ithr))).astype(jnp.int32)

        kst[sl] = keep_of(gst[sl], ist[sl], j0)
        kst[pl.ds(ntok + i * 16, 16)] = keep_of(gst2[sl], ist2[sl], j0 + 1)
        return 0

    lax.fori_loop(0, ntok // 16, cstep, 0)
    pltpu.sync_copy(kst.at[pl.ds(0, ntok)], k1_hbm.at[pl.ds(tok0, ntok)])
    pltpu.sync_copy(kst.at[pl.ds(ntok, ntok)], k2_hbm.at[pl.ds(tok0, ntok)])

    @pl.when((cid == 0) & (sid == 0))
    def _():
        pltpu.sync_copy(imp_hbm, imp_v)
        acc = jnp.zeros((16,), jnp.float32)
        for g4 in range(4):
            ev = lane16 + 16 * g4
            flat = (lax.shift_right_arithmetic(ev, 2) * 16
                    + jnp.bitwise_and(ev, 3) + 8)
            lv = plsc.load_gather(tbl_v, [flat])
            impv = imp_v[pl.ds(16 * g4, 16)]
            acc = acc + impv * lv.astype(jnp.float32)
        scale = _N_EXPERTS / (float(_N_TOKENS) * float(_N_TOKENS))
        aux_v[...] = acc * scale
        pltpu.sync_copy(aux_v, aux_hbm)


def _sc_capacity(g1b, g2b, i1v, i2v, imp):
    mesh = plsc.VectorSubcoreMesh(core_axis_name="c", subcore_axis_name="s")
    fn = pl.kernel(
        _sc_capacity_body,
        out_type=[
            jax.ShapeDtypeStruct((_N_TOKENS,), jnp.int32),
            jax.ShapeDtypeStruct((_N_TOKENS,), jnp.int32),
            jax.ShapeDtypeStruct((16,), jnp.float32),
        ],
        mesh=mesh,
        compiler_params=pltpu.CompilerParams(needs_layout_passes=False),
        scratch_types=[
            pltpu.VMEM((2 * _CS,), jnp.int32),
            pltpu.VMEM((2 * _CS,), jnp.int32),
            pltpu.VMEM((2 * _CS,), jnp.int32),
            pltpu.VMEM((2 * _CS,), jnp.int32),
            pltpu.VMEM((_NK + 64,), jnp.int32),
            pltpu.VMEM((256,), jnp.int32),
            pltpu.VMEM((16,), jnp.int32),
            pltpu.VMEM((_N_EXPERTS,), jnp.float32),
            pltpu.VMEM((2 * (_N_TOKENS // 32),), jnp.int32),
            pltpu.VMEM((16,), jnp.float32),
            pltpu.VMEM_SHARED((256,), jnp.int32),
            pltpu.SemaphoreType.DMA,
            pltpu.SemaphoreType.DMA,
            pltpu.SemaphoreType.DMA,
            pltpu.SemaphoreType.DMA,
        ],
    )
    return fn(g1b, g2b, i1v, i2v, imp)


@jax.jit
def kernel(x, W, b, training):
    del training
    n_blocks = _N_TOKENS // _TOK_BLK
    g1, g2, i1, i2, imp = pl.pallas_call(
        _router_body,
        grid=(n_blocks,),
        in_specs=[
            pl.BlockSpec((_TOK_BLK, _D_MODEL), lambda i: (i, 0)),
            pl.BlockSpec((_N_EXPERTS, _D_MODEL), lambda i: (0, 0)),
            pl.BlockSpec((_N_EXPERTS, 1), lambda i: (0, 0)),
        ],
        out_specs=[
            pl.BlockSpec((1, _TOK_BLK), lambda i: (0, i)),
            pl.BlockSpec((1, _TOK_BLK), lambda i: (0, i)),
            pl.BlockSpec((1, _TOK_BLK), lambda i: (0, i)),
            pl.BlockSpec((1, _TOK_BLK), lambda i: (0, i)),
            pl.BlockSpec((_N_EXPERTS, 1), lambda i: (0, 0)),
        ],
        out_shape=[
            jax.ShapeDtypeStruct((1, _N_TOKENS), jnp.float32),
            jax.ShapeDtypeStruct((1, _N_TOKENS), jnp.float32),
            jax.ShapeDtypeStruct((1, _N_TOKENS), jnp.int32),
            jax.ShapeDtypeStruct((1, _N_TOKENS), jnp.int32),
            jax.ShapeDtypeStruct((_N_EXPERTS, 1), jnp.float32),
        ],
        compiler_params=pltpu.CompilerParams(
            dimension_semantics=("arbitrary",)),
    )(x, W, b.reshape(_N_EXPERTS, 1))

    g1b = jax.lax.bitcast_convert_type(g1, jnp.int32).reshape(_N_TOKENS)
    g2b = jax.lax.bitcast_convert_type(g2, jnp.int32).reshape(_N_TOKENS)
    k1, k2, aux16 = _sc_capacity(
        g1b, g2b, i1.reshape(_N_TOKENS), i2.reshape(_N_TOKENS),
        imp.reshape(_N_EXPERTS))

    topk_ids = jnp.stack([i1[0], i2[0]], axis=1)
    topk_gates = jnp.stack([g1[0], g2[0]], axis=1)
    keep_mask = jnp.stack([k1, k2], axis=1) != 0
    return topk_ids, topk_gates, jnp.sum(aux16), keep_mask
